# baseline (speedup 1.0000x reference)
import functools

import jax
import jax.numpy as jnp
from jax import lax
from jax.experimental import pallas as pl
from jax.experimental.pallas import tpu as pltpu
from jax.experimental.pallas import tpu_sc as plsc

B = 16384
DIM = 64
NC = 2
NS = 16
L = 16
NW = NC * NS
BPW = B // NW
CH = 128
NCH = BPW // CH
GR = CH // L


def _body(heads_h, rels_h, tails_h, user_h, ent_h, relt_h,
          score_h, norm_h,
          hidx, ridx, tidx,
          hb0, hb1, rb0, rb1, tb0, tb1, ub0, ub1,
          sv, nv, sem0, sem1):
    wid = lax.axis_index("s") * NC + lax.axis_index("c")
    base = wid * BPW

    pltpu.sync_copy(heads_h.at[pl.ds(base, BPW)], hidx)
    pltpu.sync_copy(rels_h.at[pl.ds(base, BPW)], ridx)
    pltpu.sync_copy(tails_h.at[pl.ds(base, BPW)], tidx)
    nv[...] = jnp.zeros((L,), jnp.float32)

    hb = [hb0, hb1]
    rb = [rb0, rb1]
    tb = [tb0, tb1]
    ub = [ub0, ub1]
    sems = [sem0, sem1]

    def issue(k):
        s = k % 2
        off = k * CH
        return [
            pltpu.async_copy(ent_h.at[hidx.at[pl.ds(off, CH)]], hb[s], sems[s]),
            pltpu.async_copy(relt_h.at[ridx.at[pl.ds(off, CH)]], rb[s], sems[s]),
            pltpu.async_copy(ent_h.at[tidx.at[pl.ds(off, CH)]], tb[s], sems[s]),
            pltpu.async_copy(user_h.at[pl.ds(base + off, CH)], ub[s], sems[s]),
        ]

    def compute_chunk(h, r, t, u, k):
        def g_body(g, carry):
            row = lax.broadcasted_iota(jnp.int32, (L,), 0) + g * L
            acc = jnp.zeros((L,), jnp.float32)
            nacc = jnp.zeros((L,), jnp.float32)
            for d in range(DIM):
                col = jnp.full((L,), d, jnp.int32)
                hv = plsc.load_gather(h, [row, col])
                rv = plsc.load_gather(r, [row, col])
                tv = plsc.load_gather(t, [row, col])
                uv = plsc.load_gather(u, [row, col])
                acc = acc + (hv + rv) * uv * tv
                nacc = nacc + tv * tv
            sv[pl.ds(k * CH + g * L, L)] = acc
            nv[...] = nv[...] + jnp.maximum(nacc - 1.0, 0.0)
            return carry
        lax.fori_loop(0, GR, g_body, 0)

    pending = issue(0)
    for k in range(NCH):
        nxt = issue(k + 1) if k + 1 < NCH else None
        for hnd in pending:
            hnd.wait()
        s = k % 2
        compute_chunk(hb[s], rb[s], tb[s], ub[s], k)
        pending = nxt

    pltpu.sync_copy(sv, score_h.at[pl.ds(base, BPW)])
    pltpu.sync_copy(nv, norm_h.at[wid])


_sc_call = functools.partial(
    pl.kernel,
    mesh=plsc.VectorSubcoreMesh(core_axis_name="c", subcore_axis_name="s"),
    out_type=[
        jax.ShapeDtypeStruct((B,), jnp.float32),
        jax.ShapeDtypeStruct((NW, L), jnp.float32),
    ],
    scratch_types=[
        pltpu.VMEM((BPW,), jnp.int32),
        pltpu.VMEM((BPW,), jnp.int32),
        pltpu.VMEM((BPW,), jnp.int32),
        pltpu.VMEM((CH, DIM), jnp.float32),
        pltpu.VMEM((CH, DIM), jnp.float32),
        pltpu.VMEM((CH, DIM), jnp.float32),
        pltpu.VMEM((CH, DIM), jnp.float32),
        pltpu.VMEM((CH, DIM), jnp.float32),
        pltpu.VMEM((CH, DIM), jnp.float32),
        pltpu.VMEM((CH, DIM), jnp.float32),
        pltpu.VMEM((CH, DIM), jnp.float32),
        pltpu.VMEM((BPW,), jnp.float32),
        pltpu.VMEM((L,), jnp.float32),
        pltpu.SemaphoreType.DMA,
        pltpu.SemaphoreType.DMA,
    ],
    compiler_params=pltpu.CompilerParams(
        use_tc_tiling_on_sc=False, needs_layout_passes=False,
        skip_device_barrier=True),
)(_body)


def kernel(heads, rels, tails, e1_embedded_user, ent_table, rel_table):
    score, norm_partials = _sc_call(
        heads, rels, tails, e1_embedded_user, ent_table, rel_table)
    return score, jnp.sum(norm_partials)


# speedup vs baseline: 1.0018x; 1.0018x over previous
import functools

import jax
import jax.numpy as jnp
from jax import lax
from jax.experimental import pallas as pl
from jax.experimental.pallas import tpu as pltpu
from jax.experimental.pallas import tpu_sc as plsc

B = 16384
DIM = 64
NC = 2
NS = 16
L = 16
NW = NC * NS
BPW = B // NW
CH = 128
NCH = BPW // CH
GR = CH // L


def _body(heads_h, rels_h, tails_h, user_h, ent_h, relt_h,
          score_h, norm_h,
          hidx, ridx, tidx,
          hb0, hb1, rb0, rb1, tb0, tb1, ub0, ub1,
          sv, nv, sem0, sem1):
    wid = lax.axis_index("s") * NC + lax.axis_index("c")
    base = wid * BPW

    pltpu.sync_copy(heads_h.at[pl.ds(base, BPW)], hidx)
    pltpu.sync_copy(rels_h.at[pl.ds(base, BPW)], ridx)
    pltpu.sync_copy(tails_h.at[pl.ds(base, BPW)], tidx)
    nv[...] = jnp.zeros((L,), jnp.float32)

    hb = [hb0, hb1]
    rb = [rb0, rb1]
    tb = [tb0, tb1]
    ub = [ub0, ub1]
    sems = [sem0, sem1]

    def issue(k):
        s = k % 2
        off = k * CH
        return [
            pltpu.async_copy(ent_h.at[hidx.at[pl.ds(off, CH)]], hb[s], sems[s]),
            pltpu.async_copy(relt_h.at[ridx.at[pl.ds(off, CH)]], rb[s], sems[s]),
            pltpu.async_copy(ent_h.at[tidx.at[pl.ds(off, CH)]], tb[s], sems[s]),
            pltpu.async_copy(user_h.at[pl.ds(base + off, CH)], ub[s], sems[s]),
        ]

    def compute_chunk(h, r, t, u, k):
        def g_body(g, carry):
            row = lax.broadcasted_iota(jnp.int32, (L,), 0) + g * L

            def d_body(d, carry):
                acc, nacc = carry
                col = jnp.zeros((L,), jnp.int32) + d
                hv = plsc.load_gather(h, [row, col])
                rv = plsc.load_gather(r, [row, col])
                tv = plsc.load_gather(t, [row, col])
                uv = plsc.load_gather(u, [row, col])
                return (acc + (hv + rv) * uv * tv, nacc + tv * tv)

            acc, nacc = lax.fori_loop(
                0, DIM, d_body,
                (jnp.zeros((L,), jnp.float32), jnp.zeros((L,), jnp.float32)))
            sv[pl.ds(k * CH + g * L, L)] = acc
            nv[...] = nv[...] + jnp.maximum(nacc - 1.0, 0.0)
            return carry
        lax.fori_loop(0, GR, g_body, 0)

    pending = issue(0)
    for k in range(NCH):
        nxt = issue(k + 1) if k + 1 < NCH else None
        for hnd in pending:
            hnd.wait()
        s = k % 2
        compute_chunk(hb[s], rb[s], tb[s], ub[s], k)
        pending = nxt

    pltpu.sync_copy(sv, score_h.at[pl.ds(base, BPW)])
    pltpu.sync_copy(nv, norm_h.at[wid])


_sc_call = functools.partial(
    pl.kernel,
    mesh=plsc.VectorSubcoreMesh(core_axis_name="c", subcore_axis_name="s"),
    out_type=[
        jax.ShapeDtypeStruct((B,), jnp.float32),
        jax.ShapeDtypeStruct((NW, L), jnp.float32),
    ],
    scratch_types=[
        pltpu.VMEM((BPW,), jnp.int32),
        pltpu.VMEM((BPW,), jnp.int32),
        pltpu.VMEM((BPW,), jnp.int32),
        pltpu.VMEM((CH, DIM), jnp.float32),
        pltpu.VMEM((CH, DIM), jnp.float32),
        pltpu.VMEM((CH, DIM), jnp.float32),
        pltpu.VMEM((CH, DIM), jnp.float32),
        pltpu.VMEM((CH, DIM), jnp.float32),
        pltpu.VMEM((CH, DIM), jnp.float32),
        pltpu.VMEM((CH, DIM), jnp.float32),
        pltpu.VMEM((CH, DIM), jnp.float32),
        pltpu.VMEM((BPW,), jnp.float32),
        pltpu.VMEM((L,), jnp.float32),
        pltpu.SemaphoreType.DMA,
        pltpu.SemaphoreType.DMA,
    ],
    compiler_params=pltpu.CompilerParams(
        use_tc_tiling_on_sc=False, needs_layout_passes=False),
)(_body)


def kernel(heads, rels, tails, e1_embedded_user, ent_table, rel_table):
    score, norm_partials = _sc_call(
        heads, rels, tails, e1_embedded_user, ent_table, rel_table)
    return score, jnp.sum(norm_partials)


# speedup vs baseline: 1.0504x; 1.0485x over previous
import functools

import jax
import jax.numpy as jnp
from jax import lax
from jax.experimental import pallas as pl
from jax.experimental.pallas import tpu as pltpu
from jax.experimental.pallas import tpu_sc as plsc

B = 16384
DIM = 64
RELN = 512
NC = 2
NS = 16
L = 16
NW = NC * NS
BPW = B // NW
CH = 128
NCH = BPW // CH
GR = CH // L


def _body(heads_h, rels_h, tails_h, user_h, ent_h, relt_h,
          score_h, norm_h,
          hidx, ridx, tidx,
          relv, hb0, hb1, tb0, tb1, ub0, ub1,
          sv, nv, sem0, sem1):
    wid = lax.axis_index("s") * NC + lax.axis_index("c")
    base = wid * BPW

    pltpu.sync_copy(heads_h.at[pl.ds(base, BPW)], hidx)
    pltpu.sync_copy(rels_h.at[pl.ds(base, BPW)], ridx)
    pltpu.sync_copy(tails_h.at[pl.ds(base, BPW)], tidx)
    nv[...] = jnp.zeros((L,), jnp.float32)
    pltpu.sync_copy(relt_h, relv)

    hb = [hb0, hb1]
    tb = [tb0, tb1]
    ub = [ub0, ub1]
    sems = [sem0, sem1]

    def issue(k):
        s = k % 2
        off = k * CH
        return [
            pltpu.async_copy(ent_h.at[hidx.at[pl.ds(off, CH)]], hb[s], sems[s]),
            pltpu.async_copy(ent_h.at[tidx.at[pl.ds(off, CH)]], tb[s], sems[s]),
            pltpu.async_copy(user_h.at[:, pl.ds(base + off, CH)], ub[s], sems[s]),
        ]

    def compute_chunk(h, t, u, k):
        def g_body(g, carry):
            goff = g * L
            row = lax.broadcasted_iota(jnp.int32, (L,), 0) + goff
            rq0 = ridx[pl.ds(k * CH + goff, L)]
            acc = jnp.zeros((L,), jnp.float32)
            nacc = jnp.zeros((L,), jnp.float32)
            for d in range(DIM):
                col = jnp.full((L,), d, jnp.int32)
                hv = plsc.load_gather(h, [row, col])
                tv = plsc.load_gather(t, [row, col])
                rv = plsc.load_gather(relv, [col, rq0])
                uv = u[d, pl.ds(goff, L)]
                acc = acc + (hv + rv) * uv * tv
                nacc = nacc + tv * tv
            sv[pl.ds(k * CH + g * L, L)] = acc
            nv[...] = nv[...] + jnp.maximum(nacc - 1.0, 0.0)
            return carry
        lax.fori_loop(0, GR, g_body, 0)

    pending = issue(0)
    for k in range(NCH):
        nxt = issue(k + 1) if k + 1 < NCH else None
        for hnd in pending:
            hnd.wait()
        s = k % 2
        compute_chunk(hb[s], tb[s], ub[s], k)
        pending = nxt

    pltpu.sync_copy(sv, score_h.at[pl.ds(base, BPW)])
    pltpu.sync_copy(nv, norm_h.at[wid])


_sc_call = functools.partial(
    pl.kernel,
    mesh=plsc.VectorSubcoreMesh(core_axis_name="c", subcore_axis_name="s"),
    out_type=[
        jax.ShapeDtypeStruct((B,), jnp.float32),
        jax.ShapeDtypeStruct((NW, L), jnp.float32),
    ],
    scratch_types=[
        pltpu.VMEM((BPW,), jnp.int32),
        pltpu.VMEM((BPW,), jnp.int32),
        pltpu.VMEM((BPW,), jnp.int32),
        pltpu.VMEM((DIM, RELN), jnp.float32),
        pltpu.VMEM((CH, DIM), jnp.float32),
        pltpu.VMEM((CH, DIM), jnp.float32),
        pltpu.VMEM((CH, DIM), jnp.float32),
        pltpu.VMEM((CH, DIM), jnp.float32),
        pltpu.VMEM((DIM, CH), jnp.float32),
        pltpu.VMEM((DIM, CH), jnp.float32),
        pltpu.VMEM((BPW,), jnp.float32),
        pltpu.VMEM((L,), jnp.float32),
        pltpu.SemaphoreType.DMA,
        pltpu.SemaphoreType.DMA,
    ],
    compiler_params=pltpu.CompilerParams(
        use_tc_tiling_on_sc=False, needs_layout_passes=False),
)(_body)


def kernel(heads, rels, tails, e1_embedded_user, ent_table, rel_table):
    score, norm_partials = _sc_call(
        heads, rels, tails, e1_embedded_user.T, ent_table, rel_table.T)
    return score, jnp.sum(norm_partials)
